# baseline (speedup 1.0000x reference)
import math

import jax
import jax.numpy as jnp
import numpy as np
from jax import lax
from jax.experimental import pallas as pl
from jax.experimental.pallas import tpu as pltpu
from jax.experimental.pallas import tpu_sc as plsc

_B, _L, _D = 4, 8192, 768
_MISSING_RATE = 0.3
_KEEP = math.ceil(_L * (1.0 - _MISSING_RATE))
_R = _B * _KEEP

_NUM_TILES = 32
_CHUNK = 80
_NCHUNK = 9
_ROWS_PER_TILE = _NCHUNK * _CHUNK
_LAST = _NUM_TILES - 1
_TAIL_START = _R - _CHUNK


def _build_index_table() -> np.ndarray:
    rng = np.random.RandomState(0)
    keep = rng.choice(_L, _KEEP, replace=False)
    keep.sort()
    keep[0] = 0
    g = (np.arange(_B, dtype=np.int64)[:, None] * _L + keep[None, :]).reshape(-1)
    table = np.zeros((_NUM_TILES, _NCHUNK + 1, _CHUNK), dtype=np.int32)
    for w in range(_NUM_TILES):
        b = w * _ROWS_PER_TILE
        n = min(_ROWS_PER_TILE, _R - b)
        table[w].reshape(-1)[:n] = g[b : b + n]
    table[_LAST, 7, :] = g[_TAIL_START:_R]
    table[_LAST, 8, :] = 0
    table[_LAST, _NCHUNK, :] = np.arange(_TAIL_START, _R, dtype=np.int32)
    return table


_IDX_TABLE = _build_index_table()

_mesh = plsc.VectorSubcoreMesh(core_axis_name="c", subcore_axis_name="s")


@jax.jit
def _gather_rows(x2d: jax.Array, idx_table: jax.Array) -> jax.Array:
    @pl.kernel(
        mesh=_mesh,
        out_type=jax.ShapeDtypeStruct((_R, _D), jnp.float32),
        scratch_types=[
            pltpu.VMEM((_NCHUNK + 1, _CHUNK), jnp.int32),
            pltpu.VMEM((_CHUNK, _D), jnp.float32),
            pltpu.VMEM((_CHUNK, _D), jnp.float32),
            pltpu.SemaphoreType.DMA,
            pltpu.SemaphoreType.DMA,
            pltpu.SemaphoreType.DMA,
            pltpu.SemaphoreType.DMA,
        ],
    )
    def k(x_hbm, idx_hbm, out_hbm, idx_v, buf0, buf1, g0, g1, c0, c1):
        wid = lax.axis_index("s") * 2 + lax.axis_index("c")
        base = wid * _ROWS_PER_TILE
        is_last = wid == _LAST
        not_last = wid != _LAST
        pltpu.sync_copy(idx_hbm.at[wid], idx_v)
        bufs = (buf0, buf1)
        gsems = (g0, g1)
        csems = (c0, c1)
        gathers = [None] * _NCHUNK

        def start_gather(c):
            gathers[c] = pltpu.async_copy(
                x_hbm.at[idx_v.at[c]], bufs[c % 2], gsems[c % 2]
            )

        def linear_out(c):
            return pltpu.make_async_copy(
                bufs[c % 2], out_hbm.at[pl.ds(base + c * _CHUNK, _CHUNK)],
                csems[c % 2],
            )

        def scatter_tail():
            return pltpu.make_async_copy(
                bufs[7 % 2], out_hbm.at[idx_v.at[_NCHUNK]], csems[7 % 2]
            )

        start_gather(0)
        start_gather(1)
        for c in range(_NCHUNK):
            if c >= 2:
                linear_out(c - 2).wait()
                start_gather(c)
            gathers[c].wait()
            if c == 7:
                @pl.when(not_last)
                def _():
                    linear_out(7).start()

                @pl.when(is_last)
                def _():
                    scatter_tail().start()
            elif c == 8:
                @pl.when(not_last)
                def _():
                    linear_out(8).start()
            else:
                linear_out(c).start()

        @pl.when(not_last)
        def _():
            linear_out(7).wait()
            linear_out(8).wait()

        @pl.when(is_last)
        def _():
            scatter_tail().wait()

    return k(x2d, idx_table)


def kernel(x):
    x2d = x.reshape(_B * _L, _D)
    out = _gather_rows(x2d, jnp.asarray(_IDX_TABLE))
    return out.reshape(_B, _KEEP, _D)


# speedup vs baseline: 3.2569x; 3.2569x over previous
import math

import jax
import jax.numpy as jnp
import numpy as np
from jax import lax
from jax.experimental import pallas as pl
from jax.experimental.pallas import tpu as pltpu
from jax.experimental.pallas import tpu_sc as plsc

_B, _L, _D = 4, 8192, 768
_MISSING_RATE = 0.3
_KEEP = math.ceil(_L * (1.0 - _MISSING_RATE))

_TILES_PER_BATCH = 8
_CHUNK = 80
_NCHUNK = 9
_ROWS_PER_TILE = _NCHUNK * _CHUNK
_TAIL_START = _KEEP - _CHUNK


def _build_index_table() -> np.ndarray:
    rng = np.random.RandomState(0)
    keep = rng.choice(_L, _KEEP, replace=False)
    keep.sort()
    keep[0] = 0
    table = np.zeros((_B * _TILES_PER_BATCH, _NCHUNK + 1, _CHUNK), dtype=np.int32)
    for b in range(_B):
        src = b * _L + keep
        for t in range(_TILES_PER_BATCH):
            w = b * _TILES_PER_BATCH + t
            base = t * _ROWS_PER_TILE
            if t < _TILES_PER_BATCH - 1:
                table[w, :_NCHUNK] = src[base : base + _ROWS_PER_TILE].reshape(
                    _NCHUNK, _CHUNK
                )
            else:
                table[w].reshape(-1)[: 8 * _CHUNK] = src[base : base + 8 * _CHUNK]
                table[w, 8, :] = src[_TAIL_START:_KEEP]
                table[w, _NCHUNK, :] = np.arange(_TAIL_START, _KEEP, dtype=np.int32)
    return table


_IDX_TABLE = _build_index_table()

_mesh = plsc.VectorSubcoreMesh(core_axis_name="c", subcore_axis_name="s")


@jax.jit
def _gather_rows(x2d: jax.Array, idx_table: jax.Array) -> jax.Array:
    @pl.kernel(
        mesh=_mesh,
        out_type=jax.ShapeDtypeStruct((_B, _KEEP, _D), jnp.float32),
        scratch_types=[
            pltpu.VMEM((_NCHUNK + 1, _CHUNK), jnp.int32),
            pltpu.VMEM((_CHUNK, _D), jnp.float32),
            pltpu.VMEM((_CHUNK, _D), jnp.float32),
            pltpu.SemaphoreType.DMA,
            pltpu.SemaphoreType.DMA,
            pltpu.SemaphoreType.DMA,
            pltpu.SemaphoreType.DMA,
        ],
    )
    def k(x_hbm, idx_hbm, out_hbm, idx_v, buf0, buf1, g0, g1, c0, c1):
        wid = lax.axis_index("s") * 2 + lax.axis_index("c")
        batch = wid // _TILES_PER_BATCH
        t = wid % _TILES_PER_BATCH
        base = t * _ROWS_PER_TILE
        is_tail = t == _TILES_PER_BATCH - 1
        not_tail = t != _TILES_PER_BATCH - 1
        pltpu.sync_copy(idx_hbm.at[wid], idx_v)
        bufs = (buf0, buf1)
        gsems = (g0, g1)
        csems = (c0, c1)
        gathers = [None] * _NCHUNK

        def start_gather(c):
            gathers[c] = pltpu.async_copy(
                x_hbm.at[idx_v.at[c]], bufs[c % 2], gsems[c % 2]
            )

        def linear_out(c):
            return pltpu.make_async_copy(
                bufs[c % 2],
                out_hbm.at[batch].at[pl.ds(base + c * _CHUNK, _CHUNK)],
                csems[c % 2],
            )

        def scatter_tail():
            return pltpu.make_async_copy(
                bufs[8 % 2], out_hbm.at[batch].at[idx_v.at[_NCHUNK]], csems[8 % 2]
            )

        start_gather(0)
        start_gather(1)
        for c in range(_NCHUNK):
            if c >= 2:
                linear_out(c - 2).wait()
                start_gather(c)
            gathers[c].wait()
            if c == _NCHUNK - 1:
                @pl.when(not_tail)
                def _():
                    linear_out(c).start()

                @pl.when(is_tail)
                def _():
                    scatter_tail().start()
            else:
                linear_out(c).start()

        linear_out(_NCHUNK - 2).wait()

        @pl.when(not_tail)
        def _():
            linear_out(_NCHUNK - 1).wait()

        @pl.when(is_tail)
        def _():
            scatter_tail().wait()

    return k(x2d, idx_table)


def kernel(x):
    x2d = x.reshape(_B * _L, _D)
    return _gather_rows(x2d, jnp.asarray(_IDX_TABLE))


# speedup vs baseline: 3.2642x; 1.0023x over previous
import math

import jax
import jax.numpy as jnp
import numpy as np
from jax import lax
from jax.experimental import pallas as pl
from jax.experimental.pallas import tpu as pltpu
from jax.experimental.pallas import tpu_sc as plsc

_B, _L, _D = 4, 8192, 768
_MISSING_RATE = 0.3
_KEEP = math.ceil(_L * (1.0 - _MISSING_RATE))

_TILES_PER_BATCH = 8
_CHUNK = 48
_NCHUNK = 15
_NBUF = 3
_ROWS_PER_TILE = _NCHUNK * _CHUNK
_TAIL_START = _KEEP - _CHUNK


def _build_index_table() -> np.ndarray:
    rng = np.random.RandomState(0)
    keep = rng.choice(_L, _KEEP, replace=False)
    keep.sort()
    keep[0] = 0
    table = np.zeros((_B * _TILES_PER_BATCH, _NCHUNK + 1, _CHUNK), dtype=np.int32)
    last_rows = _KEEP - (_TILES_PER_BATCH - 1) * _ROWS_PER_TILE
    for b in range(_B):
        src = b * _L + keep
        for t in range(_TILES_PER_BATCH):
            w = b * _TILES_PER_BATCH + t
            base = t * _ROWS_PER_TILE
            if t < _TILES_PER_BATCH - 1:
                table[w, :_NCHUNK] = src[base : base + _ROWS_PER_TILE].reshape(
                    _NCHUNK, _CHUNK
                )
            else:
                n = (_NCHUNK - 1) * _CHUNK
                table[w].reshape(-1)[:n] = src[base : base + n]
                table[w, _NCHUNK - 1] = src[_TAIL_START:_KEEP]
                table[w, _NCHUNK] = np.arange(
                    _TAIL_START - base, _KEEP - base, dtype=np.int32
                )
    return table


_IDX_TABLE = _build_index_table()

_mesh = plsc.VectorSubcoreMesh(core_axis_name="c", subcore_axis_name="s")


@jax.jit
def _gather_rows(x2d: jax.Array, idx_table: jax.Array) -> jax.Array:
    @pl.kernel(
        mesh=_mesh,
        out_type=jax.ShapeDtypeStruct((_B, _KEEP, _D), jnp.float32),
        scratch_types=[
            pltpu.VMEM((_NCHUNK + 1, _CHUNK), jnp.int32),
            pltpu.VMEM((_CHUNK, _D), jnp.float32),
            pltpu.VMEM((_CHUNK, _D), jnp.float32),
            pltpu.VMEM((_CHUNK, _D), jnp.float32),
            pltpu.SemaphoreType.DMA,
            pltpu.SemaphoreType.DMA,
            pltpu.SemaphoreType.DMA,
            pltpu.SemaphoreType.DMA,
            pltpu.SemaphoreType.DMA,
            pltpu.SemaphoreType.DMA,
        ],
    )
    def k(x_hbm, idx_hbm, out_hbm, idx_v, b0, b1, b2, g0, g1, g2, c0, c1, c2):
        wid = lax.axis_index("s") * 2 + lax.axis_index("c")
        batch = wid // _TILES_PER_BATCH
        t = wid % _TILES_PER_BATCH
        base = t * _ROWS_PER_TILE
        is_tail = t == _TILES_PER_BATCH - 1
        not_tail = t != _TILES_PER_BATCH - 1
        pltpu.sync_copy(idx_hbm.at[wid], idx_v)
        bufs = (b0, b1, b2)
        gsems = (g0, g1, g2)
        csems = (c0, c1, c2)
        gathers = [None] * _NCHUNK

        def start_gather(c):
            gathers[c] = pltpu.async_copy(
                x_hbm.at[idx_v.at[c]], bufs[c % _NBUF], gsems[c % _NBUF]
            )

        def linear_out(c):
            return pltpu.make_async_copy(
                bufs[c % _NBUF],
                out_hbm.at[batch].at[pl.ds(base + c * _CHUNK, _CHUNK)],
                csems[c % _NBUF],
            )

        def scatter_tail(c):
            return pltpu.make_async_copy(
                bufs[c % _NBUF],
                out_hbm.at[batch].at[idx_v.at[_NCHUNK]],
                csems[c % _NBUF],
            )

        for c in range(_NBUF):
            start_gather(c)
        for c in range(_NCHUNK):
            if c >= _NBUF:
                linear_out(c - _NBUF).wait()
                start_gather(c)
            gathers[c].wait()
            if c == _NCHUNK - 1:
                @pl.when(not_tail)
                def _():
                    linear_out(c).start()

                @pl.when(is_tail)
                def _():
                    scatter_tail(c).start()
            else:
                linear_out(c).start()

        for c in range(_NCHUNK - _NBUF, _NCHUNK - 1):
            linear_out(c).wait()

        @pl.when(not_tail)
        def _():
            linear_out(_NCHUNK - 1).wait()

        @pl.when(is_tail)
        def _():
            scatter_tail(_NCHUNK - 1).wait()

    return k(x2d, idx_table)


def kernel(x):
    x2d = x.reshape(_B * _L, _D)
    return _gather_rows(x2d, jnp.asarray(_IDX_TABLE))


# speedup vs baseline: 3.2704x; 1.0019x over previous
import math

import jax
import jax.numpy as jnp
import numpy as np
from jax import lax
from jax.experimental import pallas as pl
from jax.experimental.pallas import tpu as pltpu
from jax.experimental.pallas import tpu_sc as plsc

_B, _L, _D = 4, 8192, 768
_MISSING_RATE = 0.3
_KEEP = math.ceil(_L * (1.0 - _MISSING_RATE))

_TILES_PER_BATCH = 8
_CHUNK = 80
_NCHUNK = 9
_ROWS_PER_TILE = _NCHUNK * _CHUNK
_TAIL_START = _KEEP - _CHUNK


def _build_index_table() -> np.ndarray:
    rng = np.random.RandomState(0)
    keep = rng.choice(_L, _KEEP, replace=False)
    keep.sort()
    keep[0] = 0
    table = np.zeros((_B * _TILES_PER_BATCH, _NCHUNK + 1, _CHUNK), dtype=np.int32)
    for b in range(_B):
        src = b * _L + keep
        for t in range(_TILES_PER_BATCH):
            w = b * _TILES_PER_BATCH + t
            base = t * _ROWS_PER_TILE
            if t < _TILES_PER_BATCH - 1:
                table[w, :_NCHUNK] = src[base : base + _ROWS_PER_TILE].reshape(
                    _NCHUNK, _CHUNK
                )
            else:
                table[w].reshape(-1)[: 8 * _CHUNK] = src[base : base + 8 * _CHUNK]
                table[w, 8, :] = src[_TAIL_START:_KEEP]
                table[w, _NCHUNK, :] = np.arange(_TAIL_START, _KEEP, dtype=np.int32)
    return table


_IDX_TABLE = _build_index_table()

_mesh = plsc.VectorSubcoreMesh(core_axis_name="c", subcore_axis_name="s")


@jax.jit
def _gather_rows(x2d: jax.Array, idx_table: jax.Array) -> jax.Array:
    @pl.kernel(
        mesh=_mesh,
        out_type=jax.ShapeDtypeStruct((_B, _KEEP, _D), jnp.float32),
        scratch_types=[
            pltpu.VMEM((_NCHUNK + 1, _CHUNK), jnp.int32),
            pltpu.VMEM((_CHUNK, _D), jnp.float32),
            pltpu.VMEM((_CHUNK, _D), jnp.float32),
            pltpu.SemaphoreType.DMA,
            pltpu.SemaphoreType.DMA,
            pltpu.SemaphoreType.DMA,
            pltpu.SemaphoreType.DMA,
        ],
    )
    def k(x_hbm, idx_hbm, out_hbm, idx_v, buf0, buf1, g0, g1, c0, c1):
        wid = lax.axis_index("s") * 2 + lax.axis_index("c")
        batch = wid // _TILES_PER_BATCH
        t = wid % _TILES_PER_BATCH
        base = t * _ROWS_PER_TILE
        is_tail = t == _TILES_PER_BATCH - 1
        not_tail = t != _TILES_PER_BATCH - 1
        pltpu.sync_copy(idx_hbm.at[wid], idx_v)
        bufs = (buf0, buf1)
        gsems = (g0, g1)
        csems = (c0, c1)
        gathers = [None] * _NCHUNK

        def start_gather(c):
            gathers[c] = pltpu.async_copy(
                x_hbm.at[idx_v.at[c]], bufs[c % 2], gsems[c % 2]
            )

        def linear_out(c):
            return pltpu.make_async_copy(
                bufs[c % 2],
                out_hbm.at[batch].at[pl.ds(base + c * _CHUNK, _CHUNK)],
                csems[c % 2],
            )

        def scatter_tail():
            return pltpu.make_async_copy(
                bufs[8 % 2], out_hbm.at[batch].at[idx_v.at[_NCHUNK]], csems[8 % 2]
            )

        start_gather(0)
        start_gather(1)
        for c in range(_NCHUNK):
            if c >= 2:
                linear_out(c - 2).wait()
                start_gather(c)
            gathers[c].wait()
            if c == _NCHUNK - 1:
                @pl.when(not_tail)
                def _():
                    linear_out(c).start()

                @pl.when(is_tail)
                def _():
                    scatter_tail().start()
            else:
                linear_out(c).start()

        linear_out(_NCHUNK - 2).wait()

        @pl.when(not_tail)
        def _():
            linear_out(_NCHUNK - 1).wait()

        @pl.when(is_tail)
        def _():
            scatter_tail().wait()

    return k(x2d, idx_table)


def kernel(x):
    x2d = x.reshape(_B * _L, _D)
    return _gather_rows(x2d, jnp.asarray(_IDX_TABLE))
